# baseline (speedup 1.0000x reference)
import math

import jax
import jax.numpy as jnp
from jax import lax
from jax.experimental import pallas as pl
from jax.experimental.pallas import tpu as pltpu
from jax.experimental.pallas import tpu_sc as plsc

D_MODEL = 64
SCALE = math.sqrt(D_MODEL)
NC, NS, L = 2, 16, 16
B_TOTAL = 4096
S_TOTAL = 200
NBUF = 4


def _tec_body(x4_hbm, tab_hbm, out_hbm, idx_v,
              i0, i1, i2, i3, g0, g1, g2, g3, t0, t1, t2, t3,
              gsem, osem):
    c = lax.axis_index("c")
    t = lax.axis_index("s")
    b0c = c * NS + t
    boff = b0c * 128

    idx2 = [i0, i1, i2, i3]
    gbufs = [g0, g1, g2, g3]
    tbufs = [t0, t1, t2, t3]

    pltpu.sync_copy(x4_hbm.at[pl.ds(b0c * S_TOTAL, S_TOTAL)], idx_v)

    rows_c = [jax.lax.iota(jnp.int32, L) + blg * L for blg in range(8)]

    def prep(g, b):
        for k in range(8):
            sl = pl.ds(k * L, L)
            idx2[b][sl] = jax.lax.shift_right_logical(idx_v[g, sl], 1)

    def gather(b):
        return pltpu.make_async_copy(tab_hbm.at[idx2[b]], gbufs[b],
                                     gsem.at[b])

    def put(g, b):
        return pltpu.make_async_copy(
            tbufs[b], out_hbm.at[g, :, pl.ds(boff, 128)], osem.at[b])

    for b in range(NBUF):
        prep(b, b)
        gather(b).start()

    niter = S_TOTAL // NBUF

    @pl.loop(0, niter)
    def chunk_loop(g0_):
        for b in range(NBUF):
            g = g0_ * NBUF + b
            gather(b).wait()

            @pl.when(g0_ > 0)
            def _wait_prev_put():
                put(g - NBUF, b).wait()

            base16 = [
                jax.lax.shift_left(idx_v[g, pl.ds(k * L, L)] & 1, 6)
                for k in range(8)
            ]
            gb = gbufs[b]
            tb = tbufs[b]

            @plsc.parallel_loop(0, D_MODEL, unroll=2)
            def _transpose(d):
                for blg in range(8):
                    cols = base16[blg] + d
                    v16 = plsc.load_gather(gb, [rows_c[blg], cols])
                    tb[d, pl.ds(blg * L, L)] = v16 * SCALE

            put(g, b).start()

            @pl.when(g0_ < niter - 1)
            def _next_gather():
                prep(g + NBUF, b)
                gather(b).start()

    for b in range(NBUF):
        put((niter - 1) * NBUF + b, b).wait()


def kernel(x, table):
    x4 = (x.T.astype(jnp.int32)
          .reshape(S_TOTAL, B_TOTAL // 128, 128)
          .transpose(1, 0, 2)
          .reshape(-1, 128))
    tab2 = table.reshape(-1, 2 * D_MODEL)

    mesh = plsc.VectorSubcoreMesh(
        core_axis_name="c", subcore_axis_name="s",
        num_cores=NC, num_subcores=NS)

    sc_call = pl.kernel(
        _tec_body,
        out_type=jax.ShapeDtypeStruct((S_TOTAL, D_MODEL, B_TOTAL),
                                      jnp.float32),
        mesh=mesh,
        scratch_types=(
            [pltpu.VMEM((S_TOTAL, 128), jnp.int32)]
            + [pltpu.VMEM((128,), jnp.int32) for _ in range(NBUF)]
            + [pltpu.VMEM((128, 128), jnp.float32) for _ in range(NBUF)]
            + [pltpu.VMEM((D_MODEL, 128), jnp.float32) for _ in range(NBUF)]
            + [pltpu.SemaphoreType.DMA((NBUF,)),
               pltpu.SemaphoreType.DMA((NBUF,))]
        ),
        compiler_params=pltpu.CompilerParams(
            use_tc_tiling_on_sc=True, needs_layout_passes=False),
    )
    out3 = sc_call(x4, tab2)
    return out3.transpose(2, 0, 1)
